# speedup vs baseline: 1.4877x; 1.4877x over previous
import functools

import jax
import jax.numpy as jnp
from jax import lax
from jax.experimental import pallas as pl
from jax.experimental.pallas import tpu as pltpu
from jax.experimental.pallas import tpu_sc as plsc

_NC = 2
_NS = 16
_NW = _NC * _NS


@functools.partial(jax.jit, static_argnums=(2, 3))
def _gather_rows(table, idx, n, d):
    b_per_w = n // _NW
    chunk = 64
    n_ch = b_per_w // chunk

    @functools.partial(
        pl.kernel,
        mesh=plsc.VectorSubcoreMesh(core_axis_name="c", subcore_axis_name="s"),
        out_type=jax.ShapeDtypeStruct((n, d), jnp.float32),
        scratch_types=[
            pltpu.VMEM((b_per_w,), jnp.int32),
            pltpu.VMEM((chunk, d), jnp.float32),
            pltpu.VMEM((chunk, d), jnp.float32),
            pltpu.SemaphoreType.DMA,
            pltpu.SemaphoreType.DMA,
        ],
    )
    def k(table_hbm, idx_hbm, out_hbm, idx_v, buf0, buf1, sem0, sem1):
        wid = lax.axis_index("s") * _NC + lax.axis_index("c")
        base = wid * b_per_w
        pltpu.sync_copy(idx_hbm.at[pl.ds(base, b_per_w)], idx_v)

        bufs = (buf0, buf1)
        sems = (sem0, sem1)
        copies = [None] * n_ch
        for c in range(n_ch):
            copies[c] = pltpu.async_copy(
                table_hbm.at[idx_v.at[pl.ds(c * chunk, chunk)]],
                bufs[c % 2],
                sems[c % 2],
            )
            if c >= 1:
                copies[c - 1].wait()
                pltpu.sync_copy(
                    bufs[(c - 1) % 2],
                    out_hbm.at[pl.ds(base + (c - 1) * chunk, chunk)],
                )
        copies[n_ch - 1].wait()
        pltpu.sync_copy(
            bufs[(n_ch - 1) % 2],
            out_hbm.at[pl.ds(base + (n_ch - 1) * chunk, chunk)],
        )

    return k(table, idx)


def kernel(inputs, embed_table):
    b, s = inputs.shape
    v, d = embed_table.shape
    n = b * s
    idx = inputs.reshape(n).astype(jnp.int32)
    out = _gather_rows(embed_table, idx, n, d)
    return out.reshape(b, s, d)
